# baseline (speedup 1.0000x reference)
import jax
import jax.numpy as jnp
from jax import lax
from jax.experimental import pallas as pl
from jax.experimental.pallas import tpu as pltpu

_T = 1024
_NC = 3
_K = _NC * _T
_D = 10000
_BD = 1000
_G = _D // _BD
_NBUF = 3
_SUP = 8

_IDX = (558, 582, 554, 552, 93, 555, 580, 571, 574, 578, 566, 287, 556, 550,
        14, 551, 64, 581)


def _body(fvals_ref, accel_ref, w_hbm, kb_ref, fw_ref, fb_ref, out_ref,
          bufs, sems, pscr):
    accel = accel_ref[...].astype(jnp.bfloat16)

    def start(i):
        pltpu.make_async_copy(
            w_hbm.at[pl.ds(i * _BD, _BD), :], bufs.at[i % _NBUF],
            sems.at[i % _NBUF]).start()

    def wait(i):
        pltpu.make_async_copy(
            w_hbm.at[pl.ds(i * _BD, _BD), :], bufs.at[i % _NBUF],
            sems.at[i % _NBUF]).wait()

    for i in range(_NBUF - 1):
        start(i)

    blk = 0
    while blk < _G:
        nb = min(_SUP, _G - blk)
        for r in range(nb):
            i = blk + r
            if i + _NBUF - 1 < _G:
                start(i + _NBUF - 1)
            wait(i)
            pscr[r:r + 1, :] = lax.dot_general(
                accel, bufs[i % _NBUF].astype(jnp.bfloat16),
                (((1,), (1,)), ((), ())),
                preferred_element_type=jnp.float32)
        rows = pl.ds(blk, nb)
        proj = pscr[0:nb, :]
        sample_hv = jnp.cos(proj + kb_ref[rows, :]) * jnp.sin(proj)

        def g(j):
            p = fvals_ref[j] * fw_ref[j, rows, :]
            return jnp.cos(p + fb_ref[j, rows, :]) * jnp.sin(p)

        feat_hv = ((g(14) + g(11)) * g(16)
                   * (g(4) + g(8) + g(6) + g(1) + g(5) + g(12) + g(17))
                   * g(13) * (g(15) + g(2)) * g(3)
                   * g(0) * g(10) * g(7) * g(9))
        out_ref[rows, :] = jnp.where(sample_hv + feat_hv > 0, 1.0, -1.0)
        blk += nb


def kernel(input, feat, kernel_w, kernel_b, feat_w, feat_b):
    accel = input[:, 1:4].T.reshape(1, _K)
    fvals = feat[jnp.array(_IDX, dtype=jnp.int32)]
    kb = kernel_b.reshape(_G, _BD)
    fw = feat_w.reshape(18, _G, _BD)
    fb = feat_b.reshape(18, _G, _BD)
    out = pl.pallas_call(
        _body,
        in_specs=[
            pl.BlockSpec(memory_space=pltpu.SMEM),
            pl.BlockSpec(memory_space=pltpu.VMEM),
            pl.BlockSpec(memory_space=pltpu.HBM),
            pl.BlockSpec(memory_space=pltpu.VMEM),
            pl.BlockSpec(memory_space=pltpu.VMEM),
            pl.BlockSpec(memory_space=pltpu.VMEM),
        ],
        out_specs=pl.BlockSpec(memory_space=pltpu.VMEM),
        out_shape=jax.ShapeDtypeStruct((_G, _BD), jnp.float32),
        scratch_shapes=[
            pltpu.VMEM((_NBUF, _BD, _K), jnp.float32),
            pltpu.SemaphoreType.DMA((_NBUF,)),
            pltpu.VMEM((_SUP, _BD), jnp.float32),
        ],
    )(fvals, accel, kernel_w, kb, fw, fb)
    return out.reshape(_D)


# speedup vs baseline: 1.0043x; 1.0043x over previous
import jax
import jax.numpy as jnp
from jax import lax
from jax.experimental import pallas as pl
from jax.experimental.pallas import tpu as pltpu

_T = 1024
_NC = 3
_K = _NC * _T
_D = 10000
_BD = 1000
_G = _D // _BD
_SUP = 8

_IDX = (558, 582, 554, 552, 93, 555, 580, 571, 574, 578, 566, 287, 556, 550,
        14, 551, 64, 581)


def _body(fvals_ref, accel_ref, w_ref, kb_ref, fw_ref, fb_ref, out_ref,
          pscr):
    i = pl.program_id(0)
    proj = lax.dot_general(
        accel_ref[...].astype(jnp.bfloat16), w_ref[...].astype(jnp.bfloat16),
        (((1,), (1,)), ((), ())),
        preferred_element_type=jnp.float32)
    pscr[pl.ds(lax.rem(i, _SUP), 1), :] = proj

    def phase2(blk, nb):
        rows = pl.ds(blk, nb)
        p_all = pscr[0:nb, :]
        sample_hv = jnp.cos(p_all + kb_ref[rows, :]) * jnp.sin(p_all)

        def g(j):
            p = fvals_ref[j] * fw_ref[j, rows, :]
            return jnp.cos(p + fb_ref[j, rows, :]) * jnp.sin(p)

        feat_hv = ((g(14) + g(11)) * g(16)
                   * (g(4) + g(8) + g(6) + g(1) + g(5) + g(12) + g(17))
                   * g(13) * (g(15) + g(2)) * g(3)
                   * g(0) * g(10) * g(7) * g(9))
        out_ref[rows, :] = jnp.where(sample_hv + feat_hv > 0, 1.0, -1.0)

    blk = 0
    while blk < _G:
        nb = min(_SUP, _G - blk)
        last = blk + nb - 1

        @pl.when(i == last)
        def _():
            phase2(blk, nb)

        blk += nb


def kernel(input, feat, kernel_w, kernel_b, feat_w, feat_b):
    accel = input[:, 1:4].T.reshape(1, _K)
    fvals = feat[jnp.array(_IDX, dtype=jnp.int32)]
    kb = kernel_b.reshape(_G, _BD)
    fw = feat_w.reshape(18, _G, _BD)
    fb = feat_b.reshape(18, _G, _BD)
    out = pl.pallas_call(
        _body,
        grid=(_G,),
        in_specs=[
            pl.BlockSpec(memory_space=pltpu.SMEM),
            pl.BlockSpec((1, _K), lambda i: (0, 0)),
            pl.BlockSpec((_BD, _K), lambda i: (i, 0)),
            pl.BlockSpec((_G, _BD), lambda i: (0, 0)),
            pl.BlockSpec((18, _G, _BD), lambda i: (0, 0, 0)),
            pl.BlockSpec((18, _G, _BD), lambda i: (0, 0, 0)),
        ],
        out_specs=pl.BlockSpec((_G, _BD), lambda i: (0, 0)),
        out_shape=jax.ShapeDtypeStruct((_G, _BD), jnp.float32),
        scratch_shapes=[
            pltpu.VMEM((_SUP, _BD), jnp.float32),
        ],
        compiler_params=pltpu.CompilerParams(
            dimension_semantics=("arbitrary",)),
    )(fvals, accel, kernel_w, kb, fw, fb)
    return out.reshape(_D)


# speedup vs baseline: 1.1399x; 1.1350x over previous
import jax
import jax.numpy as jnp
from jax import lax
from jax.experimental import pallas as pl
from jax.experimental.pallas import tpu as pltpu

_T = 1024
_NC = 3
_K = _NC * _T
_D = 10000
_BD = 1000
_G = _D // _BD

_IDX = (558, 582, 554, 552, 93, 555, 580, 571, 574, 578, 566, 287, 556, 550,
        14, 551, 64, 581)


def _body(fvals_ref, accel_ref, w_ref, kb_ref, fw_ref, fb_ref, out_ref):
    proj = lax.dot_general(
        accel_ref[...].astype(jnp.bfloat16), w_ref[...].astype(jnp.bfloat16),
        (((1,), (1,)), ((), ())),
        preferred_element_type=jnp.float32)
    p_feat = fvals_ref[...] * fw_ref[0]
    p_all = jnp.concatenate([proj, p_feat], axis=0)
    b_all = jnp.concatenate([kb_ref[0], fb_ref[0]], axis=0)
    hv = jnp.cos(p_all + b_all) * jnp.sin(p_all)

    def g(j):
        return hv[1 + j:2 + j, :]

    feat_hv = ((g(14) + g(11)) * g(16)
               * (g(4) + g(8) + g(6) + g(1) + g(5) + g(12) + g(17))
               * g(13) * (g(15) + g(2)) * g(3)
               * g(0) * g(10) * g(7) * g(9))
    out_ref[0] = jnp.where(hv[0:1, :] + feat_hv > 0, 1.0, -1.0)


def kernel(input, feat, kernel_w, kernel_b, feat_w, feat_b):
    accel = input[:, 1:4].T.reshape(1, _K)
    fvals = feat[jnp.array(_IDX, dtype=jnp.int32)].reshape(18, 1)
    kb = kernel_b.reshape(_G, 1, _BD)
    fw = feat_w.reshape(18, _G, _BD).transpose(1, 0, 2)
    fb = feat_b.reshape(18, _G, _BD).transpose(1, 0, 2)
    out = pl.pallas_call(
        _body,
        grid=(_G,),
        in_specs=[
            pl.BlockSpec((18, 1), lambda i: (0, 0)),
            pl.BlockSpec((1, _K), lambda i: (0, 0)),
            pl.BlockSpec((_BD, _K), lambda i: (i, 0)),
            pl.BlockSpec((1, 1, _BD), lambda i: (i, 0, 0)),
            pl.BlockSpec((1, 18, _BD), lambda i: (i, 0, 0)),
            pl.BlockSpec((1, 18, _BD), lambda i: (i, 0, 0)),
        ],
        out_specs=pl.BlockSpec((1, 1, _BD), lambda i: (i, 0, 0)),
        out_shape=jax.ShapeDtypeStruct((_G, 1, _BD), jnp.float32),
        compiler_params=pltpu.CompilerParams(
            dimension_semantics=("arbitrary",)),
    )(fvals, accel, kernel_w, kb, fw, fb)
    return out.reshape(_D)
